# baseline (speedup 1.0000x reference)
import functools

import jax
import jax.numpy as jnp
from jax import lax
from jax.experimental import pallas as pl
from jax.experimental.pallas import tpu as pltpu
from jax.experimental.pallas import tpu_sc as plsc

EMBED_DIM = 64
BATCH = 4096
HIST = 50

NC = 2
NS = 16
NW = NC * NS
B_PER_W = BATCH // NW
ROWS_PER_W = B_PER_W * HIST
NROW = 16
CHUNK = NROW * HIST
N_CHUNK = B_PER_W // NROW

NBUF = 2

HIST_PAD = 56
DIM_PAD = 128


def _emb_body(idx_hbm, table_hbm, out_hbm, idx_v, rows_v,
              g0, g1, w0, w1):
    wid = lax.axis_index("s") * NC + lax.axis_index("c")
    b0 = wid * B_PER_W
    pltpu.sync_copy(idx_hbm.at[wid], idx_v)

    gs = (g0, g1)
    ws = (w0, w1)

    def gather(c, b):
        return pltpu.make_async_copy(
            table_hbm.at[idx_v.at[pl.ds(c * CHUNK, CHUNK)]],
            rows_v.at[b], gs[b]
        )

    def write(c, b):
        return [
            pltpu.make_async_copy(
                rows_v.at[b, pl.ds(k * HIST, HIST)],
                out_hbm.at[b0 + c * NROW + k, pl.ds(0, HIST), pl.ds(0, EMBED_DIM)],
                ws[b],
            )
            for k in range(NROW)
        ]

    for b in range(NBUF):
        gather(b, b).start()

    def body(cc, _):
        for b in range(NBUF):
            c = cc * NBUF + b
            gather(c, b).wait()
            for cp in write(c, b):
                cp.start()
            prev_b = (b - 1) % NBUF

            @pl.when(jnp.logical_and(c >= 1, c + NBUF - 1 < N_CHUNK))
            def _():
                for cp in write(c - 1, prev_b):
                    cp.wait()
                gather(c + NBUF - 1, prev_b).start()
        return 0

    lax.fori_loop(0, N_CHUNK // NBUF, body, 0)

    for c in range(N_CHUNK - NBUF, N_CHUNK):
        for cp in write(c, c % NBUF):
            cp.wait()


@functools.partial(jax.jit)
def kernel(x, table):
    idx = x.astype(jnp.int32).reshape(NW, ROWS_PER_W)
    mesh = plsc.VectorSubcoreMesh(core_axis_name="c", subcore_axis_name="s")
    out = pl.kernel(
        _emb_body,
        out_type=jax.ShapeDtypeStruct((BATCH, HIST_PAD, DIM_PAD), jnp.float32),
        mesh=mesh,
        scratch_types=[
            pltpu.VMEM((ROWS_PER_W,), jnp.int32),
            pltpu.VMEM((NBUF, CHUNK, EMBED_DIM), jnp.float32),
        ] + [pltpu.SemaphoreType.DMA] * (2 * NBUF),
        compiler_params=pltpu.CompilerParams(use_tc_tiling_on_sc=False),
    )(idx, table)
    return out[:, :HIST, :EMBED_DIM]


# speedup vs baseline: 1.0118x; 1.0118x over previous
import functools

import jax
import jax.numpy as jnp
from jax import lax
from jax.experimental import pallas as pl
from jax.experimental.pallas import tpu as pltpu
from jax.experimental.pallas import tpu_sc as plsc

EMBED_DIM = 64
BATCH = 4096
HIST = 50

NC = 2
NS = 16
NW = NC * NS
B_PER_W = BATCH // NW
ROWS_PER_W = B_PER_W * HIST
NROW = 4
CHUNK = NROW * HIST
N_CHUNK = B_PER_W // NROW

NBUF = 8

HIST_PAD = 56
DIM_PAD = 128


def _emb_body(idx_hbm, table_hbm, out_hbm, idx_v, rows_v,
              g0, g1, g2, g3, g4, g5, g6, g7, w0, w1, w2, w3, w4, w5, w6, w7):
    wid = lax.axis_index("s") * NC + lax.axis_index("c")
    b0 = wid * B_PER_W
    pltpu.sync_copy(idx_hbm.at[wid], idx_v)

    gs = (g0, g1, g2, g3, g4, g5, g6, g7)
    ws = (w0, w1, w2, w3, w4, w5, w6, w7)

    def gather(c, b):
        return pltpu.make_async_copy(
            table_hbm.at[idx_v.at[pl.ds(c * CHUNK, CHUNK)]],
            rows_v.at[b], gs[b]
        )

    def write(c, b):
        return [
            pltpu.make_async_copy(
                rows_v.at[b, pl.ds(k * HIST, HIST)],
                out_hbm.at[b0 + c * NROW + k, pl.ds(0, HIST), pl.ds(0, EMBED_DIM)],
                ws[b],
            )
            for k in range(NROW)
        ]

    for b in range(NBUF):
        gather(b, b).start()

    def body(cc, _):
        for b in range(NBUF):
            c = cc * NBUF + b
            gather(c, b).wait()
            for cp in write(c, b):
                cp.start()
            prev_b = (b - 1) % NBUF

            @pl.when(jnp.logical_and(c >= 1, c + NBUF - 1 < N_CHUNK))
            def _():
                for cp in write(c - 1, prev_b):
                    cp.wait()
                gather(c + NBUF - 1, prev_b).start()
        return 0

    lax.fori_loop(0, N_CHUNK // NBUF, body, 0)

    for c in range(N_CHUNK - NBUF, N_CHUNK):
        for cp in write(c, c % NBUF):
            cp.wait()


@functools.partial(jax.jit)
def kernel(x, table):
    idx = x.astype(jnp.int32).reshape(NW, ROWS_PER_W)
    mesh = plsc.VectorSubcoreMesh(core_axis_name="c", subcore_axis_name="s")
    out = pl.kernel(
        _emb_body,
        out_type=jax.ShapeDtypeStruct((BATCH, HIST_PAD, DIM_PAD), jnp.float32),
        mesh=mesh,
        scratch_types=[
            pltpu.VMEM((ROWS_PER_W,), jnp.int32),
            pltpu.VMEM((NBUF, CHUNK, EMBED_DIM), jnp.float32),
        ] + [pltpu.SemaphoreType.DMA] * (2 * NBUF),
        compiler_params=pltpu.CompilerParams(use_tc_tiling_on_sc=False),
    )(idx, table)
    return out[:, :HIST, :EMBED_DIM]
